# baseline (speedup 1.0000x reference)
import jax
import jax.numpy as jnp
from jax import lax
from jax.experimental import pallas as pl
from jax.experimental.pallas import tpu as pltpu
from jax.experimental.pallas import tpu_sc as plsc

B, N, HW, D = 32, 1000, 16384, 64
NC, NS = 2, 16
NW = NC * NS
NCHUNK, CROWS = 2, 500

ROWS, LANES = (B * N * D) // 128, 128
BLK = 1600


def _gather_body(table, gidx, out, idx_v, buf_a, buf_b, sem_a, sem_b):
    wid = lax.axis_index("s") * NC + lax.axis_index("c")
    pltpu.sync_copy(gidx.at[wid], idx_v)

    bufs = (buf_a, buf_b)
    sems = (sem_a, sem_b)

    def start(j):
        pltpu.make_async_copy(table.at[idx_v.at[j]], bufs[j % 2], sems[j % 2]).start()

    start(0)
    for j in range(NCHUNK):
        if j + 1 < NCHUNK:
            start(j + 1)
        pltpu.make_async_copy(table.at[idx_v.at[j]], bufs[j % 2], sems[j % 2]).wait()
        pltpu.sync_copy(bufs[j % 2], out.at[pl.ds(wid * N + j * CROWS, CROWS)])


def _loss_body(pred_ref, targ_ref, mexp_ref, out_ref):
    @pl.when(pl.program_id(0) == 0)
    def _init():
        out_ref[...] = jnp.zeros_like(out_ref)

    p = pred_ref[...]
    t = targ_ref[...]
    m = mexp_ref[...]
    pos = t == 1.0
    arg = jnp.where(pos, p, 1.0 - p)
    w = jnp.where(pos, jnp.float32(1.5), jnp.float32(1.0))
    v = w * jnp.log(arg) * m
    out_ref[0:1, :] += jnp.sum(v, axis=0, keepdims=True)
    out_ref[1:2, :] += jnp.sum(m, axis=0, keepdims=True)


@jax.jit
def _mask_loss(table, gidx, mexp, targ):
    mesh = plsc.VectorSubcoreMesh(core_axis_name="c", subcore_axis_name="s")
    pred = pl.kernel(
        _gather_body,
        out_type=jax.ShapeDtypeStruct((B * N, D), jnp.float32),
        mesh=mesh,
        compiler_params=pltpu.CompilerParams(
            needs_layout_passes=False, use_tc_tiling_on_sc=False),
        scratch_types=[
            pltpu.VMEM((NCHUNK, CROWS), jnp.int32),
            pltpu.VMEM((CROWS, D), jnp.float32),
            pltpu.VMEM((CROWS, D), jnp.float32),
            pltpu.SemaphoreType.DMA,
            pltpu.SemaphoreType.DMA,
        ],
    )(table, gidx)

    parts = pl.pallas_call(
        _loss_body,
        grid=(ROWS // BLK,),
        in_specs=[
            pl.BlockSpec((BLK, LANES), lambda i: (i, 0)),
            pl.BlockSpec((BLK, LANES), lambda i: (i, 0)),
            pl.BlockSpec((BLK, LANES), lambda i: (i, 0)),
        ],
        out_specs=pl.BlockSpec((2, LANES), lambda i: (0, 0)),
        out_shape=jax.ShapeDtypeStruct((2, LANES), jnp.float32),
    )(pred.reshape(ROWS, LANES), targ, mexp)

    loss = 0.0 - jnp.sum(parts[0, :])
    num = jnp.sum(parts[1, :])
    return jnp.where(num > 0, loss / num, loss)


def kernel(output, mask, ind, target):
    table = output.reshape(B * HW, D)
    gidx = (ind.astype(jnp.int32) + jnp.arange(B, dtype=jnp.int32)[:, None] * HW
            ).reshape(B, NCHUNK, CROWS)
    mexp = jnp.broadcast_to(
        mask.astype(jnp.float32).reshape(B * N, 1), (B * N, D)
    ).reshape(ROWS, LANES)
    targ = target.reshape(ROWS, LANES)
    return _mask_loss(table, gidx, mexp, targ)


# speedup vs baseline: 2.2304x; 2.2304x over previous
import jax
import jax.numpy as jnp
from jax import lax
from jax.experimental import pallas as pl
from jax.experimental.pallas import tpu as pltpu
from jax.experimental.pallas import tpu_sc as plsc

B, N, HW, D = 32, 1000, 16384, 64
NC, NS = 2, 16
NW = NC * NS
K = 2
NCH = D // K
NPAD = 1008
NGRP = NPAD // 16

GB = 4


def _gather_body(table, ind, out, idx_v, buf_a, buf_b, stage, sem_a, sem_b):
    wid = lax.axis_index("s") * NC + lax.axis_index("c")

    idx_v[pl.ds(N - N % 16, 16)] = jnp.zeros((16,), jnp.int32)
    pltpu.sync_copy(ind.at[wid], idx_v.at[pl.ds(0, N)])

    bufs = (buf_a, buf_b)
    sems = (sem_a, sem_b)

    def start(c):
        pltpu.make_async_copy(
            table.at[wid, pl.ds(c * K, K)], bufs[c % 2], sems[c % 2]).start()

    start(0)
    for c in range(NCH):
        if c + 1 < NCH:
            start(c + 1)
        pltpu.make_async_copy(
            table.at[wid, pl.ds(c * K, K)], bufs[c % 2], sems[c % 2]).wait()
        rb = bufs[c % 2]

        def grp(g, _, rb=rb):
            idx16 = idx_v[pl.ds(g * 16, 16)]
            for r in range(K):
                val = plsc.load_gather(
                    rb, [jnp.full((16,), r, jnp.int32), idx16])
                stage[r, pl.ds(g * 16, 16)] = val
            return 0

        lax.fori_loop(0, NGRP, grp, 0)
        pltpu.sync_copy(stage, out.at[wid, pl.ds(c * K, K)])


def _loss_body(pred_ref, targ_ref, mask_ref, out_ref):
    @pl.when(pl.program_id(0) == 0)
    def _init():
        out_ref[...] = jnp.zeros_like(out_ref)

    p = pred_ref[:, :, :N]
    t = targ_ref[...]
    m = jnp.broadcast_to(mask_ref[...], (GB, D, N))
    pos = t == 1.0
    arg = jnp.where(pos, p, 1.0 - p)
    w = jnp.where(pos, jnp.float32(1.5), jnp.float32(1.0))
    v = w * jnp.log(arg) * m
    lane0 = lax.broadcasted_iota(jnp.int32, (1, 128), 1) == 0
    out_ref[0:1, :] += jnp.where(lane0, jnp.sum(v), 0.0)
    out_ref[1:2, :] += jnp.where(lane0, jnp.sum(m), 0.0)


@jax.jit
def _mask_loss(table_t, ind, mask3, targ_t):
    mesh = plsc.VectorSubcoreMesh(core_axis_name="c", subcore_axis_name="s")
    pred = pl.kernel(
        _gather_body,
        out_type=jax.ShapeDtypeStruct((B, D, NPAD), jnp.float32),
        mesh=mesh,
        compiler_params=pltpu.CompilerParams(
            needs_layout_passes=False, use_tc_tiling_on_sc=False),
        scratch_types=[
            pltpu.VMEM((NPAD,), jnp.int32),
            pltpu.VMEM((K, HW), jnp.float32),
            pltpu.VMEM((K, HW), jnp.float32),
            pltpu.VMEM((K, NPAD), jnp.float32),
            pltpu.SemaphoreType.DMA,
            pltpu.SemaphoreType.DMA,
        ],
    )(table_t, ind)

    parts = pl.pallas_call(
        _loss_body,
        grid=(B // GB,),
        in_specs=[
            pl.BlockSpec((GB, D, NPAD), lambda i: (i, 0, 0)),
            pl.BlockSpec((GB, D, N), lambda i: (i, 0, 0)),
            pl.BlockSpec((GB, 1, N), lambda i: (i, 0, 0)),
        ],
        out_specs=pl.BlockSpec((2, 128), lambda i: (0, 0)),
        out_shape=jax.ShapeDtypeStruct((2, 128), jnp.float32),
    )(pred, targ_t, mask3)

    loss = 0.0 - jnp.sum(parts[0, :])
    num = jnp.sum(parts[1, :])
    return jnp.where(num > 0, loss / num, loss)


def kernel(output, mask, ind, target):
    table_t = output.transpose(0, 2, 1)
    targ_t = target.transpose(0, 2, 3, 1).reshape(B, D, N)
    mask3 = mask.astype(jnp.float32).reshape(B, 1, N)
    return _mask_loss(table_t, ind.astype(jnp.int32), mask3, targ_t)


# speedup vs baseline: 4.6303x; 2.0760x over previous
import jax
import jax.numpy as jnp
from jax import lax
from jax.experimental import pallas as pl
from jax.experimental.pallas import tpu as pltpu
from jax.experimental.pallas import tpu_sc as plsc

B, N, HW, D = 32, 1000, 16384, 64
NC, NS = 2, 16
NW = NC * NS
NPAD = 1024
NCH = 32

GB = 4


def _gather_body(table, ind, out, idx_v, buf_a, buf_b, stage, sem_a, sem_b):
    wid = lax.axis_index("s") * NC + lax.axis_index("c")

    for nb in range(8):
        pltpu.sync_copy(ind.at[wid, nb], idx_v.at[pl.ds(nb * 128, 128)])

    bufs = (buf_a, buf_b)
    sems = (sem_a, sem_b)

    def start(c):
        db, rr = c // 4, c % 4
        pltpu.make_async_copy(
            table.at[wid, db, :, pl.ds(rr * 2, 2), :],
            bufs[c % 2], sems[c % 2]).start()

    start(0)
    for c in range(NCH):
        if c + 1 < NCH:
            start(c + 1)
        db, rr = c // 4, c % 4
        pltpu.make_async_copy(
            table.at[wid, db, :, pl.ds(rr * 2, 2), :],
            bufs[c % 2], sems[c % 2]).wait()
        rb = bufs[c % 2]

        def grp(g, _, rb=rb):
            h = idx_v[pl.ds(g * 16, 16)]
            hb = jnp.right_shift(h, 7)
            hl = jnp.bitwise_and(h, 127)
            nb = g // 8
            lo = (g % 8) * 16
            for r in range(2):
                val = plsc.load_gather(
                    rb, [hb, jnp.full((16,), r, jnp.int32), hl])
                stage[nb, r, pl.ds(lo, 16)] = val
            return 0

        lax.fori_loop(0, 64, grp, 0)
        pltpu.sync_copy(stage, out.at[wid, db, :, pl.ds(rr * 2, 2), :])


def _loss_body(pred_ref, targ_ref, mask_ref, out_ref):
    @pl.when(pl.program_id(0) == 0)
    def _init():
        out_ref[...] = jnp.zeros_like(out_ref)

    p = pred_ref[:, :, :N]
    t = targ_ref[...]
    m = jnp.broadcast_to(mask_ref[...], (GB, D, N))
    pos = t == 1.0
    arg = jnp.where(pos, p, 1.0 - p)
    w = jnp.where(pos, jnp.float32(1.5), jnp.float32(1.0))
    v = w * jnp.log(arg) * m
    lane0 = lax.broadcasted_iota(jnp.int32, (1, 128), 1) == 0
    out_ref[0:1, :] += jnp.where(lane0, jnp.sum(v), 0.0)
    out_ref[1:2, :] += jnp.where(lane0, jnp.sum(m), 0.0)


@jax.jit
def _mask_loss(table5, ind5, mask3, targ_t):
    mesh = plsc.VectorSubcoreMesh(core_axis_name="c", subcore_axis_name="s")
    pred5 = pl.kernel(
        _gather_body,
        out_type=jax.ShapeDtypeStruct((B, 8, 8, 8, 128), jnp.float32),
        mesh=mesh,
        compiler_params=pltpu.CompilerParams(
            needs_layout_passes=False, use_tc_tiling_on_sc=True),
        scratch_types=[
            pltpu.VMEM((NPAD,), jnp.int32),
            pltpu.VMEM((128, 2, 128), jnp.float32),
            pltpu.VMEM((128, 2, 128), jnp.float32),
            pltpu.VMEM((8, 2, 128), jnp.float32),
            pltpu.SemaphoreType.DMA,
            pltpu.SemaphoreType.DMA,
        ],
    )(table5, ind5)

    pred3 = pred5.transpose(0, 1, 3, 2, 4).reshape(B, D, NPAD)

    parts = pl.pallas_call(
        _loss_body,
        grid=(B // GB,),
        in_specs=[
            pl.BlockSpec((GB, D, NPAD), lambda i: (i, 0, 0)),
            pl.BlockSpec((GB, D, N), lambda i: (i, 0, 0)),
            pl.BlockSpec((GB, 1, N), lambda i: (i, 0, 0)),
        ],
        out_specs=pl.BlockSpec((2, 128), lambda i: (0, 0)),
        out_shape=jax.ShapeDtypeStruct((2, 128), jnp.float32),
    )(pred3, targ_t, mask3)

    loss = 0.0 - jnp.sum(parts[0, :])
    num = jnp.sum(parts[1, :])
    return jnp.where(num > 0, loss / num, loss)


def kernel(output, mask, ind, target):
    table5 = output.reshape(B, 128, 128, 8, 8).transpose(0, 3, 1, 4, 2)
    ind5 = jnp.pad(ind.astype(jnp.int32), ((0, 0), (0, NPAD - N))
                   ).reshape(B, 8, 128)
    targ_t = target.transpose(0, 2, 3, 1).reshape(B, D, N)
    mask3 = mask.astype(jnp.float32).reshape(B, 1, N)
    return _mask_loss(table5, ind5, mask3, targ_t)
